# baseline (speedup 1.0000x reference)
import functools

import jax
import jax.numpy as jnp
from jax import lax
from jax.experimental import pallas as pl
from jax.experimental.pallas import tpu as pltpu
from jax.experimental.pallas import tpu_sc as plsc

N = 50000
C = 128
K = 27
NP = 50176
BATCH = 128
GRP = 4
NC, NS = 2, 16
NW = NC * NS
CH = 8
R = 6272
RPAD = R + 128
ZR = RPAD // NS
BM = 512
MAXB = K * (NP // BM)
BD = 2000

_mesh = plsc.VectorSubcoreMesh(
    core_axis_name="c", subcore_axis_name="s", num_cores=NC, num_subcores=NS)


def _sload(tab_v, idx):
    row = (idx // 16) * 16
    lane = idx % 16
    v = tab_v[pl.ds(row, 16)]
    li = lax.broadcasted_iota(jnp.int32, (16,), 0)
    return jnp.max(jnp.where(li == lane, v, 0))


@functools.partial(
    pl.kernel,
    out_type=jax.ShapeDtypeStruct((K, NP, C), jnp.float32),
    mesh=_mesh,
    scratch_types=[
        pltpu.VMEM((GRP * BATCH,), jnp.int32),
        pltpu.VMEM((GRP * BATCH, C), jnp.float32),
        pltpu.VMEM((32,), jnp.int32),
        pltpu.SemaphoreType.DMA,
        pltpu.SemaphoreType.DMA,
        pltpu.SemaphoreType.DMA,
    ],
    compiler_params=pltpu.CompilerParams(needs_layout_passes=False),
)
def _gather_phase(feats, srcp, tab_a, g, idx_v, rows_v, tab_v,
                  sem_i, sem_g, sem_w):
    cid = lax.axis_index("c")
    sid = lax.axis_index("s")
    wid = sid * NC + cid
    pltpu.sync_copy(tab_a, tab_v)
    tot = _sload(tab_v, K)
    share = (tot + NW - 1) // NW
    q0 = wid * share
    q1 = jnp.minimum(q0 + share, tot)

    def _do(kk, o0, a, gn):
        lp0 = (a - o0) * BATCH
        flat = kk * NP + lp0
        pltpu.async_copy(srcp.at[pl.ds(flat, gn * BATCH)],
                         idx_v.at[pl.ds(0, gn * BATCH)], sem_i).wait()
        for j in range(gn * BATCH // 16):
            sl = pl.ds(j * 16, 16)
            idx_v[sl] = jnp.minimum(idx_v[sl], N - 1)
        cps = [pltpu.async_copy(feats.at[idx_v.at[pl.ds(i * BATCH, BATCH)]],
                                rows_v.at[pl.ds(i * BATCH, BATCH)], sem_g)
               for i in range(gn)]
        for cp in cps:
            cp.wait()
        pltpu.async_copy(rows_v.at[pl.ds(0, gn * BATCH)],
                         g.at[kk, pl.ds(lp0, gn * BATCH)], sem_w).wait()

    def kk_body(kk, carry):
        o0 = _sload(tab_v, kk)
        o1 = _sload(tab_v, kk + 1)
        a0 = jnp.maximum(q0, o0)
        a1 = jnp.minimum(q1, o1)
        n = jnp.maximum(0, a1 - a0)
        ng = n // GRP

        def g_body(gi, c2):
            _do(kk, o0, a0 + gi * GRP, GRP)
            return c2

        def r_body(ri, c2):
            _do(kk, o0, a0 + ng * GRP + ri, 1)
            return c2

        lax.fori_loop(0, ng, g_body, 0)
        lax.fori_loop(0, n - ng * GRP, r_body, 0)
        return carry

    lax.fori_loop(0, K, kk_body, 0)


@functools.partial(
    pl.kernel,
    out_type=jax.ShapeDtypeStruct((CH * R, C), jnp.float32),
    mesh=_mesh,
    scratch_types=[
        pltpu.VMEM((GRP * BATCH,), jnp.int32),
        pltpu.VMEM((GRP, BATCH), jnp.int32),
        pltpu.VMEM((GRP * BATCH, C), jnp.float32),
        pltpu.VMEM((3 * CH * 32,), jnp.int32),
        pltpu.VMEM_SHARED((RPAD, C), jnp.float32),
        pltpu.SemaphoreType.DMA,
        pltpu.SemaphoreType.DMA,
        pltpu.SemaphoreType.DMA,
    ],
    compiler_params=pltpu.CompilerParams(needs_layout_passes=False),
)
def _scatter_phase(p, dstp, ctab, zrows, out,
                   didx_v, rel_v, prow_v, tab_v, acc, sem_i, sem_r, sem_s):
    cid = lax.axis_index("c")
    sid = lax.axis_index("s")
    li = lax.broadcasted_iota(jnp.int32, (16,), 0)
    pltpu.sync_copy(ctab, tab_v)

    for ci in range(CH // NC):
        chunk = cid + ci * NC
        base = chunk * R
        pltpu.sync_copy(zrows, acc.at[pl.ds(sid * ZR, ZR)])
        plsc.subcore_barrier()
        tot = _sload(tab_v, chunk * 32 + K)
        share = (tot + NS - 1) // NS
        q0 = sid * share
        q1 = jnp.minimum(q0 + share, tot)

        def kk_body(kk, carry):
            o0 = _sload(tab_v, chunk * 32 + kk)
            o1 = _sload(tab_v, chunk * 32 + kk + 1)
            s = _sload(tab_v, CH * 32 + chunk * 32 + kk)
            e = _sload(tab_v, 2 * CH * 32 + chunk * 32 + kk)
            s8 = (s // 8) * 8
            a0 = jnp.maximum(q0, o0)
            a1 = jnp.minimum(q1, o1)
            n = jnp.maximum(0, a1 - a0)
            ng = n // GRP

            def _do(a, gn):
                p0 = s8 + (a - o0) * BATCH
                flat = kk * NP + p0
                cpi = pltpu.async_copy(dstp.at[pl.ds(flat, gn * BATCH)],
                                       didx_v.at[pl.ds(0, gn * BATCH)], sem_i)
                cpr = pltpu.async_copy(p.at[kk, pl.ds(p0, gn * BATCH)],
                                       prow_v.at[pl.ds(0, gn * BATCH)], sem_r)
                cpi.wait()
                for i in range(gn):
                    for j in range(BATCH // 16):
                        sl = pl.ds(j * 16, 16)
                        pp = p0 + i * BATCH + j * 16 + li
                        ok = (pp >= s) & (pp < e)
                        rel_v[i, sl] = jnp.where(
                            ok, didx_v[pl.ds(i * BATCH + j * 16, 16)] - base, R)
                cpr.wait()
                cps = [pltpu.async_copy(prow_v.at[pl.ds(i * BATCH, BATCH)],
                                        acc.at[rel_v.at[i]], sem_s, add=True)
                       for i in range(gn)]
                for cp in cps:
                    cp.wait()

            def g_body(gi, c2):
                _do(a0 + gi * GRP, GRP)
                return c2

            def r_body(ri, c2):
                _do(a0 + ng * GRP + ri, 1)
                return c2

            lax.fori_loop(0, ng, g_body, 0)
            lax.fori_loop(0, n - ng * GRP, r_body, 0)
            return carry

        lax.fori_loop(0, K, kk_body, 0)
        plsc.subcore_barrier()
        WR = R // NS
        pltpu.sync_copy(acc.at[pl.ds(sid * WR, WR)],
                        out.at[pl.ds(base + sid * WR, WR)])
        plsc.subcore_barrier()


def _mm_body(kk_tab, b_tab, g, w, pout):
    pout[...] = jnp.dot(g[0], w[0], preferred_element_type=jnp.float32)[None]


def _fin_body(acc_b, feats_b, w13_b, bias_b, o_b):
    o_b[...] = jnp.maximum(
        acc_b[...]
        + jnp.dot(feats_b[...], w13_b[...], preferred_element_type=jnp.float32)
        + bias_b[...],
        0.0,
    )


def kernel(in_feats, weight, bias, pairs_in, pairs_out, in_coors):
    pi = pairs_in.astype(jnp.int32)
    po = pairs_out.astype(jnp.int32)
    cnt = jnp.sum(pi < N, axis=1, dtype=jnp.int32)

    src_pad = jnp.pad(po, ((0, 0), (0, NP - N)), constant_values=N).reshape(-1)
    dst_pad = jnp.pad(pi, ((0, 0), (0, NP - N)), constant_values=N).reshape(-1)

    nba = ((cnt + BATCH - 1) // BATCH).at[13].set(0)
    offa = jnp.cumsum(nba) - nba
    tab_a = (jnp.zeros((32,), jnp.int32).at[:K].set(offa.astype(jnp.int32))
             .at[K].set(jnp.sum(nba, dtype=jnp.int32)))

    bounds_lo = (jnp.arange(CH, dtype=jnp.int32) * R)
    bounds_hi = jnp.minimum(bounds_lo + R, N)
    ss_lo = jax.vmap(
        lambda r: jnp.searchsorted(r, bounds_lo).astype(jnp.int32))(pi)
    ss_hi = jax.vmap(
        lambda r: jnp.searchsorted(r, bounds_hi).astype(jnp.int32))(pi)
    s8c = (ss_lo // 8) * 8
    nbc = jnp.where(ss_hi > ss_lo,
                    (ss_hi - s8c + BATCH - 1) // BATCH, 0).at[13, :].set(0)
    ccum = jnp.cumsum(nbc, axis=0) - nbc

    def _pack(block27):
        b = jnp.zeros((32, CH), jnp.int32).at[:K].set(block27.astype(jnp.int32))
        return b.T.reshape(-1)

    ctab = jnp.concatenate([
        _pack(ccum).at[jnp.arange(CH) * 32 + K].set(
            jnp.sum(nbc, axis=0, dtype=jnp.int32)),
        _pack(ss_lo),
        _pack(ss_hi),
    ])

    zrows = jnp.zeros((ZR, C), jnp.float32)
    wsym = weight[::-1]
    nbb = ((cnt + BM - 1) // BM).at[13].set(0).astype(jnp.int32)
    nblocks = jnp.sum(nbb)
    offs = jnp.cumsum(nbb) - nbb
    kk_tab = jnp.repeat(jnp.arange(K, dtype=jnp.int32), nbb,
                        total_repeat_length=MAXB)
    b_tab = (jnp.arange(MAXB, dtype=jnp.int32)
             - jnp.repeat(offs.astype(jnp.int32), nbb,
                          total_repeat_length=MAXB))

    g_arr = _gather_phase(in_feats, src_pad, tab_a)

    p_arr = pl.pallas_call(
        _mm_body,
        grid_spec=pltpu.PrefetchScalarGridSpec(
            num_scalar_prefetch=2,
            grid=(nblocks,),
            in_specs=[
                pl.BlockSpec((1, BM, C), lambda i, kt, bt: (kt[i], bt[i], 0)),
                pl.BlockSpec((1, C, C), lambda i, kt, bt: (kt[i], 0, 0)),
            ],
            out_specs=pl.BlockSpec((1, BM, C),
                                   lambda i, kt, bt: (kt[i], bt[i], 0)),
        ),
        out_shape=jax.ShapeDtypeStruct((K, NP, C), jnp.float32),
    )(kk_tab, b_tab, g_arr, wsym)

    acc_arr = _scatter_phase(p_arr, dst_pad, ctab, zrows)

    out_feats = pl.pallas_call(
        _fin_body,
        grid=(N // BD,),
        in_specs=[
            pl.BlockSpec((BD, C), lambda b: (b, 0)),
            pl.BlockSpec((BD, C), lambda b: (b, 0)),
            pl.BlockSpec((C, C), lambda b: (0, 0)),
            pl.BlockSpec((1, C), lambda b: (0, 0)),
        ],
        out_specs=pl.BlockSpec((BD, C), lambda b: (b, 0)),
        out_shape=jax.ShapeDtypeStruct((N, C), jnp.float32),
    )(acc_arr, in_feats, weight[13], bias.reshape(1, C))

    return (out_feats, in_coors)


# speedup vs baseline: 1.4581x; 1.4581x over previous
import functools

import jax
import jax.numpy as jnp
from jax import lax
from jax.experimental import pallas as pl
from jax.experimental.pallas import tpu as pltpu
from jax.experimental.pallas import tpu_sc as plsc

N = 50000
C = 128
K = 27
NP = 50176
BATCH = 128
GRP = 4
NC, NS = 2, 16
NW = NC * NS
CH = 8
R = 6272
RPAD = R + 128
ZR = RPAD // NS
BM = 1024
MAXB = K * (NP // BM)
BD = 2000

_mesh = plsc.VectorSubcoreMesh(
    core_axis_name="c", subcore_axis_name="s", num_cores=NC, num_subcores=NS)


def _sload(tab_v, idx):
    row = (idx // 16) * 16
    lane = idx % 16
    v = tab_v[pl.ds(row, 16)]
    li = lax.broadcasted_iota(jnp.int32, (16,), 0)
    return jnp.max(jnp.where(li == lane, v, 0))


@functools.partial(
    pl.kernel,
    out_type=jax.ShapeDtypeStruct((K, NP, C), jnp.float32),
    mesh=_mesh,
    scratch_types=[
        pltpu.VMEM((GRP * BATCH,), jnp.int32),
        pltpu.VMEM((GRP * BATCH, C), jnp.float32),
        pltpu.VMEM((32,), jnp.int32),
        pltpu.SemaphoreType.DMA,
        pltpu.SemaphoreType.DMA,
        pltpu.SemaphoreType.DMA,
    ],
    compiler_params=pltpu.CompilerParams(needs_layout_passes=False),
)
def _gather_phase(feats, srcp, tab_a, g, idx_v, rows_v, tab_v,
                  sem_i, sem_g, sem_w):
    cid = lax.axis_index("c")
    sid = lax.axis_index("s")
    wid = sid * NC + cid
    pltpu.sync_copy(tab_a, tab_v)
    tot = _sload(tab_v, K)
    share = (tot + NW - 1) // NW
    q0 = wid * share
    q1 = jnp.minimum(q0 + share, tot)

    def _do(kk, o0, a, gn):
        lp0 = (a - o0) * BATCH
        flat = kk * NP + lp0
        pltpu.async_copy(srcp.at[pl.ds(flat, gn * BATCH)],
                         idx_v.at[pl.ds(0, gn * BATCH)], sem_i).wait()
        for j in range(gn * BATCH // 16):
            sl = pl.ds(j * 16, 16)
            idx_v[sl] = jnp.minimum(idx_v[sl], N - 1)
        cps = [pltpu.async_copy(feats.at[idx_v.at[pl.ds(i * BATCH, BATCH)]],
                                rows_v.at[pl.ds(i * BATCH, BATCH)], sem_g)
               for i in range(gn)]
        for cp in cps:
            cp.wait()
        pltpu.async_copy(rows_v.at[pl.ds(0, gn * BATCH)],
                         g.at[kk, pl.ds(lp0, gn * BATCH)], sem_w).wait()

    def kk_body(kk, carry):
        o0 = _sload(tab_v, kk)
        o1 = _sload(tab_v, kk + 1)
        a0 = jnp.maximum(q0, o0)
        a1 = jnp.minimum(q1, o1)
        n = jnp.maximum(0, a1 - a0)
        ng = n // GRP

        def g_body(gi, c2):
            _do(kk, o0, a0 + gi * GRP, GRP)
            return c2

        def r_body(ri, c2):
            _do(kk, o0, a0 + ng * GRP + ri, 1)
            return c2

        lax.fori_loop(0, ng, g_body, 0)
        lax.fori_loop(0, n - ng * GRP, r_body, 0)
        return carry

    lax.fori_loop(0, K, kk_body, 0)


@functools.partial(
    pl.kernel,
    out_type=jax.ShapeDtypeStruct((CH * R, C), jnp.float32),
    mesh=_mesh,
    scratch_types=[
        pltpu.VMEM((GRP * BATCH,), jnp.int32),
        pltpu.VMEM((GRP, BATCH), jnp.int32),
        pltpu.VMEM((GRP * BATCH, C), jnp.float32),
        pltpu.VMEM((3 * CH * 32,), jnp.int32),
        pltpu.VMEM_SHARED((RPAD, C), jnp.float32),
        pltpu.SemaphoreType.DMA,
        pltpu.SemaphoreType.DMA,
        pltpu.SemaphoreType.DMA,
    ],
    compiler_params=pltpu.CompilerParams(needs_layout_passes=False),
)
def _scatter_phase(p, dstp, ctab, zrows, out,
                   didx_v, rel_v, prow_v, tab_v, acc, sem_i, sem_r, sem_s):
    cid = lax.axis_index("c")
    sid = lax.axis_index("s")
    li = lax.broadcasted_iota(jnp.int32, (16,), 0)
    pltpu.sync_copy(ctab, tab_v)

    for ci in range(CH // NC):
        chunk = cid + ci * NC
        base = chunk * R
        pltpu.sync_copy(zrows, acc.at[pl.ds(sid * ZR, ZR)])
        plsc.subcore_barrier()
        tot = _sload(tab_v, chunk * 32 + K)
        share = (tot + NS - 1) // NS
        q0 = sid * share
        q1 = jnp.minimum(q0 + share, tot)

        def kk_body(kk, carry):
            o0 = _sload(tab_v, chunk * 32 + kk)
            o1 = _sload(tab_v, chunk * 32 + kk + 1)
            s = _sload(tab_v, CH * 32 + chunk * 32 + kk)
            e = _sload(tab_v, 2 * CH * 32 + chunk * 32 + kk)
            s8 = (s // 8) * 8
            a0 = jnp.maximum(q0, o0)
            a1 = jnp.minimum(q1, o1)
            n = jnp.maximum(0, a1 - a0)
            ng = n // GRP

            def _do(a, gn):
                p0 = s8 + (a - o0) * BATCH
                flat = kk * NP + p0
                cpi = pltpu.async_copy(dstp.at[pl.ds(flat, gn * BATCH)],
                                       didx_v.at[pl.ds(0, gn * BATCH)], sem_i)
                cpr = pltpu.async_copy(p.at[kk, pl.ds(p0, gn * BATCH)],
                                       prow_v.at[pl.ds(0, gn * BATCH)], sem_r)
                cpi.wait()
                for i in range(gn):
                    for j in range(BATCH // 16):
                        sl = pl.ds(j * 16, 16)
                        pp = p0 + i * BATCH + j * 16 + li
                        ok = (pp >= s) & (pp < e)
                        rel_v[i, sl] = jnp.where(
                            ok, didx_v[pl.ds(i * BATCH + j * 16, 16)] - base, R)
                cpr.wait()
                cps = [pltpu.async_copy(prow_v.at[pl.ds(i * BATCH, BATCH)],
                                        acc.at[rel_v.at[i]], sem_s, add=True)
                       for i in range(gn)]
                for cp in cps:
                    cp.wait()

            def g_body(gi, c2):
                _do(a0 + gi * GRP, GRP)
                return c2

            def r_body(ri, c2):
                _do(a0 + ng * GRP + ri, 1)
                return c2

            lax.fori_loop(0, ng, g_body, 0)
            lax.fori_loop(0, n - ng * GRP, r_body, 0)
            return carry

        lax.fori_loop(0, K, kk_body, 0)
        plsc.subcore_barrier()
        WR = R // NS
        pltpu.sync_copy(acc.at[pl.ds(sid * WR, WR)],
                        out.at[pl.ds(base + sid * WR, WR)])
        plsc.subcore_barrier()


def _mm_body(kk_tab, b_tab, g, w, pout):
    pout[...] = jnp.dot(g[0], w[0], preferred_element_type=jnp.float32)[None]


def _fin_body(acc_b, feats_b, w13_b, bias_b, o_b):
    o_b[...] = jnp.maximum(
        acc_b[...]
        + jnp.dot(feats_b[...], w13_b[...], preferred_element_type=jnp.float32)
        + bias_b[...],
        0.0,
    )


def kernel(in_feats, weight, bias, pairs_in, pairs_out, in_coors):
    pi = pairs_in.astype(jnp.int32)
    po = pairs_out.astype(jnp.int32)
    bounds_all = jnp.minimum(
        jnp.arange(1, CH + 1, dtype=jnp.int32) * R, N)
    ss_all = jnp.sum(pi[:, :, None] < bounds_all[None, None, :], axis=1,
                     dtype=jnp.int32)
    cnt = ss_all[:, -1]

    src_pad = jnp.pad(po, ((0, 0), (0, NP - N)), constant_values=N).reshape(-1)
    dst_pad = jnp.pad(pi, ((0, 0), (0, NP - N)), constant_values=N).reshape(-1)

    nba = ((cnt + BATCH - 1) // BATCH).at[13].set(0)
    offa = jnp.cumsum(nba) - nba
    tab_a = (jnp.zeros((32,), jnp.int32).at[:K].set(offa.astype(jnp.int32))
             .at[K].set(jnp.sum(nba, dtype=jnp.int32)))

    ss_lo = jnp.concatenate(
        [jnp.zeros((K, 1), jnp.int32), ss_all[:, :-1]], axis=1)
    ss_hi = ss_all
    s8c = (ss_lo // 8) * 8
    nbc = jnp.where(ss_hi > ss_lo,
                    (ss_hi - s8c + BATCH - 1) // BATCH, 0).at[13, :].set(0)
    ccum = jnp.cumsum(nbc, axis=0) - nbc

    def _pack(block27):
        b = jnp.zeros((32, CH), jnp.int32).at[:K].set(block27.astype(jnp.int32))
        return b.T.reshape(-1)

    ctab = jnp.concatenate([
        _pack(ccum).at[jnp.arange(CH) * 32 + K].set(
            jnp.sum(nbc, axis=0, dtype=jnp.int32)),
        _pack(ss_lo),
        _pack(ss_hi),
    ])

    zrows = jnp.zeros((ZR, C), jnp.float32)
    wsym = weight[::-1]
    nbb = ((cnt + BM - 1) // BM).at[13].set(0).astype(jnp.int32)
    nblocks = jnp.sum(nbb)
    offs = jnp.cumsum(nbb) - nbb
    kk_tab = jnp.repeat(jnp.arange(K, dtype=jnp.int32), nbb,
                        total_repeat_length=MAXB)
    b_tab = (jnp.arange(MAXB, dtype=jnp.int32)
             - jnp.repeat(offs.astype(jnp.int32), nbb,
                          total_repeat_length=MAXB))

    g_arr = _gather_phase(in_feats, src_pad, tab_a)

    p_arr = pl.pallas_call(
        _mm_body,
        grid_spec=pltpu.PrefetchScalarGridSpec(
            num_scalar_prefetch=2,
            grid=(nblocks,),
            in_specs=[
                pl.BlockSpec((1, BM, C), lambda i, kt, bt: (kt[i], bt[i], 0)),
                pl.BlockSpec((1, C, C), lambda i, kt, bt: (kt[i], 0, 0)),
            ],
            out_specs=pl.BlockSpec((1, BM, C),
                                   lambda i, kt, bt: (kt[i], bt[i], 0)),
        ),
        out_shape=jax.ShapeDtypeStruct((K, NP, C), jnp.float32),
    )(kk_tab, b_tab, g_arr, wsym)

    acc_arr = _scatter_phase(p_arr, dst_pad, ctab, zrows)

    out_feats = pl.pallas_call(
        _fin_body,
        grid=(N // BD,),
        in_specs=[
            pl.BlockSpec((BD, C), lambda b: (b, 0)),
            pl.BlockSpec((BD, C), lambda b: (b, 0)),
            pl.BlockSpec((C, C), lambda b: (0, 0)),
            pl.BlockSpec((1, C), lambda b: (0, 0)),
        ],
        out_specs=pl.BlockSpec((BD, C), lambda b: (b, 0)),
        out_shape=jax.ShapeDtypeStruct((N, C), jnp.float32),
    )(acc_arr, in_feats, weight[13], bias.reshape(1, C))

    return (out_feats, in_coors)


# speedup vs baseline: 1.4638x; 1.0040x over previous
import functools

import jax
import jax.numpy as jnp
from jax import lax
from jax.experimental import pallas as pl
from jax.experimental.pallas import tpu as pltpu
from jax.experimental.pallas import tpu_sc as plsc

N = 50000
C = 128
K = 27
NP = 50176
BATCH = 128
GRP = 4
NC, NS = 2, 16
NW = NC * NS
CH = 8
R = 6272
RPAD = R + 128
ZR = RPAD // NS
BM = 1024
MAXB = K * (NP // BM)
BD = 2000

_mesh = plsc.VectorSubcoreMesh(
    core_axis_name="c", subcore_axis_name="s", num_cores=NC, num_subcores=NS)


def _sload(tab_v, idx):
    row = (idx // 16) * 16
    lane = idx % 16
    v = tab_v[pl.ds(row, 16)]
    li = lax.broadcasted_iota(jnp.int32, (16,), 0)
    return jnp.max(jnp.where(li == lane, v, 0))


@functools.partial(
    pl.kernel,
    out_type=jax.ShapeDtypeStruct((K, NP, C), jnp.float32),
    mesh=_mesh,
    scratch_types=[
        pltpu.VMEM((GRP * BATCH,), jnp.int32),
        pltpu.VMEM((GRP * BATCH, C), jnp.float32),
        pltpu.VMEM((32,), jnp.int32),
        pltpu.SemaphoreType.DMA,
        pltpu.SemaphoreType.DMA,
        pltpu.SemaphoreType.DMA,
    ],
    compiler_params=pltpu.CompilerParams(needs_layout_passes=False),
)
def _gather_phase(feats, srcp, tab_a, g, idx_v, rows_v, tab_v,
                  sem_i, sem_g, sem_w):
    cid = lax.axis_index("c")
    sid = lax.axis_index("s")
    wid = sid * NC + cid
    pltpu.sync_copy(tab_a, tab_v)
    tot = _sload(tab_v, K)
    share = (tot + NW - 1) // NW
    q0 = wid * share
    q1 = jnp.minimum(q0 + share, tot)

    def _do(kk, o0, a, gn):
        lp0 = (a - o0) * BATCH
        flat = kk * NP + lp0
        pltpu.async_copy(srcp.at[pl.ds(flat, gn * BATCH)],
                         idx_v.at[pl.ds(0, gn * BATCH)], sem_i).wait()
        for j in range(gn * BATCH // 16):
            sl = pl.ds(j * 16, 16)
            idx_v[sl] = jnp.minimum(idx_v[sl], N - 1)
        cps = [pltpu.async_copy(feats.at[idx_v.at[pl.ds(i * BATCH, BATCH)]],
                                rows_v.at[pl.ds(i * BATCH, BATCH)], sem_g)
               for i in range(gn)]
        for cp in cps:
            cp.wait()
        pltpu.async_copy(rows_v.at[pl.ds(0, gn * BATCH)],
                         g.at[kk, pl.ds(lp0, gn * BATCH)], sem_w).wait()

    def kk_body(kk, carry):
        o0 = _sload(tab_v, kk)
        o1 = _sload(tab_v, kk + 1)
        a0 = jnp.maximum(q0, o0)
        a1 = jnp.minimum(q1, o1)
        n = jnp.maximum(0, a1 - a0)
        ng = n // GRP

        def g_body(gi, c2):
            _do(kk, o0, a0 + gi * GRP, GRP)
            return c2

        def r_body(ri, c2):
            _do(kk, o0, a0 + ng * GRP + ri, 1)
            return c2

        lax.fori_loop(0, ng, g_body, 0)
        lax.fori_loop(0, n - ng * GRP, r_body, 0)
        return carry

    lax.fori_loop(0, K, kk_body, 0)


@functools.partial(
    pl.kernel,
    out_type=jax.ShapeDtypeStruct((CH * R, C), jnp.float32),
    mesh=_mesh,
    scratch_types=[
        pltpu.VMEM((GRP * BATCH,), jnp.int32),
        pltpu.VMEM((GRP, BATCH), jnp.int32),
        pltpu.VMEM((GRP * BATCH, C), jnp.float32),
        pltpu.VMEM((3 * CH * 32,), jnp.int32),
        pltpu.VMEM_SHARED((RPAD, C), jnp.float32),
        pltpu.SemaphoreType.DMA,
        pltpu.SemaphoreType.DMA,
        pltpu.SemaphoreType.DMA,
    ],
    compiler_params=pltpu.CompilerParams(needs_layout_passes=False),
)
def _scatter_phase(p, dstp, ctab, zrows, out,
                   didx_v, rel_v, prow_v, tab_v, acc, sem_i, sem_r, sem_s):
    cid = lax.axis_index("c")
    sid = lax.axis_index("s")
    li = lax.broadcasted_iota(jnp.int32, (16,), 0)
    pltpu.sync_copy(ctab, tab_v)

    for ci in range(CH // NC):
        chunk = cid + ci * NC
        base = chunk * R
        pltpu.sync_copy(zrows, acc.at[pl.ds(sid * ZR, ZR)])
        plsc.subcore_barrier()
        tot = _sload(tab_v, chunk * 32 + K)
        share = (tot + NS - 1) // NS
        q0 = sid * share
        q1 = jnp.minimum(q0 + share, tot)

        def kk_body(kk, carry):
            o0 = _sload(tab_v, chunk * 32 + kk)
            o1 = _sload(tab_v, chunk * 32 + kk + 1)
            s = _sload(tab_v, CH * 32 + chunk * 32 + kk)
            e = _sload(tab_v, 2 * CH * 32 + chunk * 32 + kk)
            s8 = (s // 8) * 8
            a0 = jnp.maximum(q0, o0)
            a1 = jnp.minimum(q1, o1)
            n = jnp.maximum(0, a1 - a0)
            ng = n // GRP

            def _do(a, gn):
                p0 = s8 + (a - o0) * BATCH
                flat = kk * NP + p0
                cpi = pltpu.async_copy(dstp.at[pl.ds(flat, gn * BATCH)],
                                       didx_v.at[pl.ds(0, gn * BATCH)], sem_i)
                cpr = pltpu.async_copy(p.at[kk, pl.ds(p0, gn * BATCH)],
                                       prow_v.at[pl.ds(0, gn * BATCH)], sem_r)
                cpi.wait()
                for i in range(gn):
                    for j in range(BATCH // 16):
                        sl = pl.ds(j * 16, 16)
                        pp = p0 + i * BATCH + j * 16 + li
                        ok = (pp >= s) & (pp < e)
                        rel_v[i, sl] = jnp.where(
                            ok, didx_v[pl.ds(i * BATCH + j * 16, 16)] - base, R)
                cpr.wait()
                cps = [pltpu.async_copy(prow_v.at[pl.ds(i * BATCH, BATCH)],
                                        acc.at[rel_v.at[i]], sem_s, add=True)
                       for i in range(gn)]
                for cp in cps:
                    cp.wait()

            def g_body(gi, c2):
                _do(a0 + gi * GRP, GRP)
                return c2

            def r_body(ri, c2):
                _do(a0 + ng * GRP + ri, 1)
                return c2

            lax.fori_loop(0, ng, g_body, 0)
            lax.fori_loop(0, n - ng * GRP, r_body, 0)
            return carry

        lax.fori_loop(0, K, kk_body, 0)
        plsc.subcore_barrier()
        WR = R // NS
        pltpu.sync_copy(acc.at[pl.ds(sid * WR, WR)],
                        out.at[pl.ds(base + sid * WR, WR)])
        plsc.subcore_barrier()


def _mm_body(kk_tab, b_tab, g, w, pout):
    pout[...] = jnp.dot(g[0], w[0], preferred_element_type=jnp.float32)[None]


def _fin_body(acc1_b, acc2_b, feats_b, w13_b, bias_b, o_b):
    o_b[...] = jnp.maximum(
        acc1_b[...] + acc2_b[...]
        + jnp.dot(feats_b[...], w13_b[...], preferred_element_type=jnp.float32)
        + bias_b[...],
        0.0,
    )


def kernel(in_feats, weight, bias, pairs_in, pairs_out, in_coors):
    pi = pairs_in.astype(jnp.int32)
    po = pairs_out.astype(jnp.int32)
    bounds_all = jnp.minimum(
        jnp.arange(1, CH + 1, dtype=jnp.int32) * R, N)
    ss_all = jnp.sum(pi[:, :, None] < bounds_all[None, None, :], axis=1,
                     dtype=jnp.int32)
    cnt = ss_all[:, -1]

    src_pad = jnp.pad(po, ((0, 0), (0, NP - N)), constant_values=N).reshape(-1)
    dst_pad = jnp.pad(pi, ((0, 0), (0, NP - N)), constant_values=N).reshape(-1)

    ss_lo = jnp.concatenate(
        [jnp.zeros((K, 1), jnp.int32), ss_all[:, :-1]], axis=1)
    ss_hi = ss_all
    s8c = (ss_lo // 8) * 8
    zrows = jnp.zeros((ZR, C), jnp.float32)
    wsym = weight[::-1]
    kmask1 = jnp.arange(K, dtype=jnp.int32) < 13
    kmask2 = jnp.arange(K, dtype=jnp.int32) > 13

    def _tables(kmask):
        nba = jnp.where(kmask, (cnt + BATCH - 1) // BATCH, 0)
        offa = jnp.cumsum(nba) - nba
        tab_a = (jnp.zeros((32,), jnp.int32)
                 .at[:K].set(offa.astype(jnp.int32))
                 .at[K].set(jnp.sum(nba, dtype=jnp.int32)))

        nbc = jnp.where(kmask[:, None] & (ss_hi > ss_lo),
                        (ss_hi - s8c + BATCH - 1) // BATCH, 0)
        ccum = jnp.cumsum(nbc, axis=0) - nbc

        def _pack(block27):
            b = jnp.zeros((32, CH), jnp.int32).at[:K].set(
                block27.astype(jnp.int32))
            return b.T.reshape(-1)

        ctab = jnp.concatenate([
            _pack(ccum).at[jnp.arange(CH) * 32 + K].set(
                jnp.sum(nbc, axis=0, dtype=jnp.int32)),
            _pack(ss_lo),
            _pack(ss_hi),
        ])

        nbb = jnp.where(kmask, (cnt + BM - 1) // BM, 0).astype(jnp.int32)
        nblocks = jnp.sum(nbb)
        offs = jnp.cumsum(nbb) - nbb
        kk_tab = jnp.repeat(jnp.arange(K, dtype=jnp.int32), nbb,
                            total_repeat_length=MAXB)
        b_tab = (jnp.arange(MAXB, dtype=jnp.int32)
                 - jnp.repeat(offs.astype(jnp.int32), nbb,
                              total_repeat_length=MAXB))
        return tab_a, ctab, nblocks, kk_tab, b_tab

    def _mm_call(nblocks, kk_tab, b_tab, g_arr):
        return pl.pallas_call(
            _mm_body,
            grid_spec=pltpu.PrefetchScalarGridSpec(
                num_scalar_prefetch=2,
                grid=(nblocks,),
                in_specs=[
                    pl.BlockSpec((1, BM, C),
                                 lambda i, kt, bt: (kt[i], bt[i], 0)),
                    pl.BlockSpec((1, C, C), lambda i, kt, bt: (kt[i], 0, 0)),
                ],
                out_specs=pl.BlockSpec((1, BM, C),
                                       lambda i, kt, bt: (kt[i], bt[i], 0)),
            ),
            out_shape=jax.ShapeDtypeStruct((K, NP, C), jnp.float32),
        )(kk_tab, b_tab, g_arr, wsym)

    tab_a1, ctab1, nblocks1, kk_tab1, b_tab1 = _tables(kmask1)
    tab_a2, ctab2, nblocks2, kk_tab2, b_tab2 = _tables(kmask2)

    g1 = _gather_phase(in_feats, src_pad, tab_a1)
    g2 = _gather_phase(in_feats, src_pad, tab_a2)
    p1 = _mm_call(nblocks1, kk_tab1, b_tab1, g1)
    p2 = _mm_call(nblocks2, kk_tab2, b_tab2, g2)
    acc1 = _scatter_phase(p1, dst_pad, ctab1, zrows)
    acc2 = _scatter_phase(p2, dst_pad, ctab2, zrows)

    out_feats = pl.pallas_call(
        _fin_body,
        grid=(N // BD,),
        in_specs=[
            pl.BlockSpec((BD, C), lambda b: (b, 0)),
            pl.BlockSpec((BD, C), lambda b: (b, 0)),
            pl.BlockSpec((BD, C), lambda b: (b, 0)),
            pl.BlockSpec((C, C), lambda b: (0, 0)),
            pl.BlockSpec((1, C), lambda b: (0, 0)),
        ],
        out_specs=pl.BlockSpec((BD, C), lambda b: (b, 0)),
        out_shape=jax.ShapeDtypeStruct((N, C), jnp.float32),
    )(acc1, acc2, in_feats, weight[13], bias.reshape(1, C))

    return (out_feats, in_coors)


# speedup vs baseline: 1.5517x; 1.0600x over previous
import functools

import jax
import jax.numpy as jnp
from jax import lax
from jax.experimental import pallas as pl
from jax.experimental.pallas import tpu as pltpu
from jax.experimental.pallas import tpu_sc as plsc

N = 50000
C = 128
K = 27
NP = 50176
BATCH = 128
GRP = 4
NC, NS = 2, 16
NW = NC * NS
CH = 8
R = 6272
RPAD = R + 128
ZR = RPAD // NS
ZB = 80
BM = 1024
MAXB = K * (NP // BM)
BD = 2000

_mesh = plsc.VectorSubcoreMesh(
    core_axis_name="c", subcore_axis_name="s", num_cores=NC, num_subcores=NS)


def _sload(tab_v, idx):
    row = (idx // 16) * 16
    lane = idx % 16
    v = tab_v[pl.ds(row, 16)]
    li = lax.broadcasted_iota(jnp.int32, (16,), 0)
    return jnp.max(jnp.where(li == lane, v, 0))


@functools.partial(
    pl.kernel,
    out_type=jax.ShapeDtypeStruct((K, NP, C), jnp.float32),
    mesh=_mesh,
    scratch_types=[
        pltpu.VMEM((GRP * BATCH,), jnp.int32),
        pltpu.VMEM((GRP * BATCH, C), jnp.float32),
        pltpu.VMEM((32,), jnp.int32),
        pltpu.SemaphoreType.DMA,
        pltpu.SemaphoreType.DMA,
        pltpu.SemaphoreType.DMA,
    ],
    compiler_params=pltpu.CompilerParams(needs_layout_passes=False),
)
def _gather_phase(feats, srcp, tab_a, g, idx_v, rows_v, tab_v,
                  sem_i, sem_g, sem_w):
    cid = lax.axis_index("c")
    sid = lax.axis_index("s")
    wid = sid * NC + cid
    pltpu.sync_copy(tab_a, tab_v)
    lo = _sload(tab_v, K + 1)
    hi = _sload(tab_v, K + 2)
    share = (hi - lo + NW - 1) // NW
    q0 = lo + wid * share
    q1 = jnp.minimum(q0 + share, hi)

    def _do(kk, o0, a, gn):
        lp0 = (a - o0) * BATCH
        flat = kk * NP + lp0
        pltpu.async_copy(srcp.at[pl.ds(flat, gn * BATCH)],
                         idx_v.at[pl.ds(0, gn * BATCH)], sem_i).wait()
        for j in range(gn * BATCH // 16):
            sl = pl.ds(j * 16, 16)
            idx_v[sl] = jnp.minimum(idx_v[sl], N - 1)
        cps = [pltpu.async_copy(feats.at[idx_v.at[pl.ds(i * BATCH, BATCH)]],
                                rows_v.at[pl.ds(i * BATCH, BATCH)], sem_g)
               for i in range(gn)]
        for cp in cps:
            cp.wait()
        pltpu.async_copy(rows_v.at[pl.ds(0, gn * BATCH)],
                         g.at[kk, pl.ds(lp0, gn * BATCH)], sem_w).wait()

    def kk_body(kk, carry):
        o0 = _sload(tab_v, kk)
        o1 = _sload(tab_v, kk + 1)
        a0 = jnp.maximum(q0, o0)
        a1 = jnp.minimum(q1, o1)
        n = jnp.maximum(0, a1 - a0)
        ng = n // GRP

        def g_body(gi, c2):
            _do(kk, o0, a0 + gi * GRP, GRP)
            return c2

        def r_body(ri, c2):
            _do(kk, o0, a0 + ng * GRP + ri, 1)
            return c2

        lax.fori_loop(0, ng, g_body, 0)
        lax.fori_loop(0, n - ng * GRP, r_body, 0)
        return carry

    lax.fori_loop(0, K, kk_body, 0)


@functools.partial(
    pl.kernel,
    out_type=jax.ShapeDtypeStruct((CH * R, C), jnp.float32),
    mesh=_mesh,
    scratch_types=[
        pltpu.VMEM((GRP * BATCH,), jnp.int32),
        pltpu.VMEM((GRP, BATCH), jnp.int32),
        pltpu.VMEM((GRP * BATCH, C), jnp.float32),
        pltpu.VMEM((3 * CH * 32,), jnp.int32),
        pltpu.VMEM((ZB, C), jnp.float32),
        pltpu.VMEM_SHARED((RPAD, C), jnp.float32),
        pltpu.SemaphoreType.DMA,
        pltpu.SemaphoreType.DMA,
        pltpu.SemaphoreType.DMA,
    ],
    compiler_params=pltpu.CompilerParams(needs_layout_passes=False),
)
def _scatter_phase(p, dstp, ctab, zrows, out,
                   didx_v, rel_v, prow_v, tab_v, zero_v, acc,
                   sem_i, sem_r, sem_s):
    cid = lax.axis_index("c")
    sid = lax.axis_index("s")
    li = lax.broadcasted_iota(jnp.int32, (16,), 0)
    pltpu.sync_copy(ctab, tab_v)

    for ci in range(CH // NC):
        chunk = cid + ci * NC
        base = chunk * R
        pltpu.sync_copy(zrows, acc.at[pl.ds(sid * ZR, ZR)])
        plsc.subcore_barrier()
        tot = _sload(tab_v, chunk * 32 + K)
        share = (tot + NS - 1) // NS
        q0 = sid * share
        q1 = jnp.minimum(q0 + share, tot)

        def kk_body(kk, carry):
            o0 = _sload(tab_v, chunk * 32 + kk)
            o1 = _sload(tab_v, chunk * 32 + kk + 1)
            s = _sload(tab_v, CH * 32 + chunk * 32 + kk)
            e = _sload(tab_v, 2 * CH * 32 + chunk * 32 + kk)
            s8 = (s // 8) * 8
            a0 = jnp.maximum(q0, o0)
            a1 = jnp.minimum(q1, o1)
            n = jnp.maximum(0, a1 - a0)
            ng = n // GRP

            def _do(a, gn):
                p0 = s8 + (a - o0) * BATCH
                flat = kk * NP + p0
                cpi = pltpu.async_copy(dstp.at[pl.ds(flat, gn * BATCH)],
                                       didx_v.at[pl.ds(0, gn * BATCH)], sem_i)
                cpr = pltpu.async_copy(p.at[kk, pl.ds(p0, gn * BATCH)],
                                       prow_v.at[pl.ds(0, gn * BATCH)], sem_r)
                cpi.wait()
                for i in range(gn):
                    for j in range(BATCH // 16):
                        sl = pl.ds(j * 16, 16)
                        pp = p0 + i * BATCH + j * 16 + li
                        ok = (pp >= s) & (pp < e)
                        rel_v[i, sl] = jnp.where(
                            ok, didx_v[pl.ds(i * BATCH + j * 16, 16)] - base, R)
                cpr.wait()
                cps = [pltpu.async_copy(prow_v.at[pl.ds(i * BATCH, BATCH)],
                                        acc.at[rel_v.at[i]], sem_s, add=True)
                       for i in range(gn)]
                for cp in cps:
                    cp.wait()

            def g_body(gi, c2):
                _do(a0 + gi * GRP, GRP)
                return c2

            def r_body(ri, c2):
                _do(a0 + ng * GRP + ri, 1)
                return c2

            lax.fori_loop(0, ng, g_body, 0)
            lax.fori_loop(0, n - ng * GRP, r_body, 0)
            return carry

        lax.fori_loop(0, K, kk_body, 0)
        plsc.subcore_barrier()
        WR = R // NS
        pltpu.sync_copy(acc.at[pl.ds(sid * WR, WR)],
                        out.at[pl.ds(base + sid * WR, WR)])
        plsc.subcore_barrier()


def _mm_body(kk_tab, b_tab, g, w, pout):
    pout[...] = jnp.dot(g[0], w[0], preferred_element_type=jnp.float32)[None]


def _mm_body2(kk_tab, b_tab, p_prev, g, w, pout):
    pout[...] = jnp.dot(g[0], w[0], preferred_element_type=jnp.float32)[None]


def _fin_body(acc_b, feats_b, w13_b, bias_b, o_b):
    o_b[...] = jnp.maximum(
        acc_b[...]
        + jnp.dot(feats_b[...], w13_b[...], preferred_element_type=jnp.float32)
        + bias_b[...],
        0.0,
    )


def kernel(in_feats, weight, bias, pairs_in, pairs_out, in_coors):
    pi = pairs_in.astype(jnp.int32)
    po = pairs_out.astype(jnp.int32)
    bounds_all = jnp.minimum(
        jnp.arange(1, CH + 1, dtype=jnp.int32) * R, N)
    ss_all = jnp.sum(pi[:, :, None] < bounds_all[None, None, :], axis=1,
                     dtype=jnp.int32)
    cnt = ss_all[:, -1]

    src_pad = jnp.pad(po, ((0, 0), (0, NP - N)), constant_values=N).reshape(-1)
    dst_pad = jnp.pad(pi, ((0, 0), (0, NP - N)), constant_values=N).reshape(-1)

    ss_lo = jnp.concatenate(
        [jnp.zeros((K, 1), jnp.int32), ss_all[:, :-1]], axis=1)
    ss_hi = ss_all
    s8c = (ss_lo // 8) * 8
    zrows = jnp.zeros((ZR, C), jnp.float32)
    wsym = weight[::-1]

    nba = ((cnt + BATCH - 1) // BATCH).at[13].set(0)
    offa = (jnp.cumsum(nba) - nba).astype(jnp.int32)
    tota = jnp.sum(nba, dtype=jnp.int32)
    cut_a = offa[13]
    tab_base = jnp.zeros((32,), jnp.int32).at[:K].set(offa).at[K].set(tota)
    tab_a1 = tab_base.at[K + 1].set(0).at[K + 2].set(cut_a)
    tab_a2 = tab_base.at[K + 1].set(cut_a).at[K + 2].set(tota)

    nbc = jnp.where(ss_hi > ss_lo,
                    (ss_hi - s8c + BATCH - 1) // BATCH, 0).at[13, :].set(0)
    ccum = jnp.cumsum(nbc, axis=0) - nbc

    def _pack(block27):
        b = jnp.zeros((32, CH), jnp.int32).at[:K].set(block27.astype(jnp.int32))
        return b.T.reshape(-1)

    ctab = jnp.concatenate([
        _pack(ccum).at[jnp.arange(CH) * 32 + K].set(
            jnp.sum(nbc, axis=0, dtype=jnp.int32)),
        _pack(ss_lo),
        _pack(ss_hi),
    ])

    nbb = ((cnt + BM - 1) // BM).at[13].set(0).astype(jnp.int32)
    offs = jnp.cumsum(nbb) - nbb
    nblocks1 = offs[14]
    nblocks2 = jnp.sum(nbb) - nblocks1
    kk_tab = jnp.repeat(jnp.arange(K, dtype=jnp.int32), nbb,
                        total_repeat_length=MAXB)
    b_tab = (jnp.arange(MAXB, dtype=jnp.int32)
             - jnp.repeat(offs.astype(jnp.int32), nbb,
                          total_repeat_length=MAXB))
    kk_tab2 = jnp.roll(kk_tab, -nblocks1)
    b_tab2 = jnp.roll(b_tab, -nblocks1)

    def _mm_call(nblocks, kt, bt, g_arr, p_prev=None):
        args = [kt, bt]
        in_specs = []
        aliases = {}
        if p_prev is not None:
            args.append(p_prev)
            in_specs.append(pl.BlockSpec(memory_space=pltpu.MemorySpace.HBM))
            aliases = {2: 0}
        args += [g_arr, wsym]
        in_specs += [
            pl.BlockSpec((1, BM, C), lambda i, kt, bt: (kt[i], bt[i], 0)),
            pl.BlockSpec((1, C, C), lambda i, kt, bt: (kt[i], 0, 0)),
        ]
        body = _mm_body2 if p_prev is not None else _mm_body
        return pl.pallas_call(
            body,
            grid_spec=pltpu.PrefetchScalarGridSpec(
                num_scalar_prefetch=2,
                grid=(nblocks,),
                in_specs=in_specs,
                out_specs=pl.BlockSpec((1, BM, C),
                                       lambda i, kt, bt: (kt[i], bt[i], 0)),
            ),
            out_shape=jax.ShapeDtypeStruct((K, NP, C), jnp.float32),
            input_output_aliases=aliases,
        )(*args)

    g1 = _gather_phase(in_feats, src_pad, tab_a1)
    g2 = _gather_phase(in_feats, src_pad, tab_a2)
    p1 = _mm_call(nblocks1, kk_tab, b_tab, g1)
    p2 = _mm_call(nblocks2, kk_tab2, b_tab2, g2, p_prev=p1)

    acc_arr = _scatter_phase(p2, dst_pad, ctab, zrows)

    out_feats = pl.pallas_call(
        _fin_body,
        grid=(N // BD,),
        in_specs=[
            pl.BlockSpec((BD, C), lambda b: (b, 0)),
            pl.BlockSpec((BD, C), lambda b: (b, 0)),
            pl.BlockSpec((C, C), lambda b: (0, 0)),
            pl.BlockSpec((1, C), lambda b: (0, 0)),
        ],
        out_specs=pl.BlockSpec((BD, C), lambda b: (b, 0)),
        out_shape=jax.ShapeDtypeStruct((N, C), jnp.float32),
    )(acc_arr, in_feats, weight[13], bias.reshape(1, C))

    return (out_feats, in_coors)
